# baseline (speedup 1.0000x reference)
import jax
import jax.numpy as jnp
from jax.experimental import pallas as pl
from jax.experimental.pallas import tpu as pltpu
from jax.experimental.pallas import tpu_sc as plsc

_D = 32
_W = 128
_K = 8


def _gather_kernel(num_indices):
    mesh = plsc.VectorSubcoreMesh(core_axis_name="c", subcore_axis_name="s")
    num_rows = num_indices // _W

    @jax.jit
    def run(table, idx_flat):
        idx2d = idx_flat.reshape(num_rows, _W)

        @pl.kernel(
            out_type=jax.ShapeDtypeStruct((num_indices, _D), jnp.float32),
            mesh=mesh,
            scratch_types=[pltpu.SemaphoreType.DMA],
            compiler_params=pltpu.CompilerParams(use_tc_tiling_on_sc=False),
        )
        def k(table_hbm, i_hbm, o_hbm, sem):
            def body(i_vmem, o_vmem):
                copies = [
                    pltpu.async_copy(
                        table_hbm.at[i_vmem.at[j]],
                        o_vmem.at[pl.ds(j * _W, _W)],
                        sem,
                    )
                    for j in range(_K)
                ]
                for c in copies:
                    c.wait()

            pltpu.emit_pipeline(
                body,
                grid=(num_rows // _K,),
                in_specs=[pl.BlockSpec((_K, _W), lambda i: (i, 0))],
                out_specs=[pl.BlockSpec((_K * _W, _D), lambda i: (i, 0))],
                core_axis_name=("c", "s"),
                dimension_semantics=(pltpu.PARALLEL,),
            )(i_hbm, o_hbm)

        return k(table, idx2d)

    return run


def kernel(input_text, embedding_table):
    batch, seq = input_text.shape
    n = batch * seq
    run = _gather_kernel(n)
    out = run(embedding_table, input_text.reshape(n))
    return out.reshape(batch, seq, _D)


# speedup vs baseline: 1.0188x; 1.0188x over previous
import jax
import jax.numpy as jnp
from jax import lax
from jax.experimental import pallas as pl
from jax.experimental.pallas import tpu as pltpu
from jax.experimental.pallas import tpu_sc as plsc

_D = 32
_W = 128
_K = 8
_L = 16


def _build(batch, seq):
    n = batch * seq
    num_rows = n // _W
    mesh = plsc.VectorSubcoreMesh(core_axis_name="c", subcore_axis_name="s")

    @jax.jit
    def run(table, input_text):
        idx2d = input_text.T.reshape(num_rows, _W)

        @pl.kernel(
            out_type=jax.ShapeDtypeStruct((n, _D), jnp.float32),
            mesh=mesh,
            scratch_types=[pltpu.SemaphoreType.DMA],
            compiler_params=pltpu.CompilerParams(use_tc_tiling_on_sc=False),
        )
        def gather_k(table_hbm, i_hbm, y_hbm, sem):
            def body(i_vmem, y_vmem):
                copies = [
                    pltpu.async_copy(
                        table_hbm.at[i_vmem.at[j]],
                        y_vmem.at[pl.ds(j * _W, _W)],
                        sem,
                    )
                    for j in range(_K)
                ]
                for c in copies:
                    c.wait()

            pltpu.emit_pipeline(
                body,
                grid=(num_rows // _K,),
                in_specs=[pl.BlockSpec((_K, _W), lambda i: (i, 0))],
                out_specs=[pl.BlockSpec((_K * _W, _D), lambda i: (i, 0))],
                core_axis_name=("c", "s"),
                dimension_semantics=(pltpu.PARALLEL,),
            )(i_hbm, y_hbm)

        y = gather_k(table, idx2d).reshape(n * _D)

        @pl.kernel(
            out_type=jax.ShapeDtypeStruct((seq, _D, batch), jnp.float32),
            mesh=mesh,
            compiler_params=pltpu.CompilerParams(
                use_tc_tiling_on_sc=True, needs_layout_passes=False
            ),
        )
        def transpose_k(y_hbm, x_hbm):
            def body(y_vmem, x_vmem):
                base = lax.iota(jnp.int32, _L) * _D

                @pl.loop(0, _D)
                def _(d):
                    @pl.loop(0, _W // _L)
                    def _(j):
                        vals = plsc.load_gather(
                            y_vmem, [base + (j * (_L * _D) + d)]
                        )
                        x_vmem[0, d, pl.ds(j * _L, _L)] = vals

            pltpu.emit_pipeline(
                body,
                grid=(num_rows,),
                in_specs=[pl.BlockSpec((_W * _D,), lambda i: (i,))],
                out_specs=[
                    pl.BlockSpec(
                        (1, _D, _W),
                        lambda i: (i // (batch // _W), 0, i % (batch // _W)),
                    )
                ],
                core_axis_name=("c", "s"),
                dimension_semantics=(pltpu.PARALLEL,),
            )(y_hbm, x_hbm)

        x = transpose_k(y)
        return jnp.transpose(x, (2, 0, 1))

    return run


def kernel(input_text, embedding_table):
    batch, seq = input_text.shape
    run = _build(batch, seq)
    return run(embedding_table, input_text)
